# Initial kernel scaffold; baseline (speedup 1.0000x reference)
import jax
import jax.numpy as jnp
from jax.experimental import pallas as pl


def kernel(gt_vertices, est_vertices, gt_connections, est_connections):
    raise NotImplementedError("write your pallas kernel here")


# speedup vs baseline: 9.1338x; 9.1338x over previous
import jax
import jax.numpy as jnp
from jax import lax
from jax.experimental import pallas as pl
from jax.experimental.pallas import tpu as pltpu
from jax.experimental.pallas import tpu_sc as plsc

_LANES = 16
_NC = 2
_NS = 16
_NW = _NC * _NS
_CHUNK = 128
_E_PAD = 819200
_PER_TILE = _E_PAD // _NW
_ITERS = _PER_TILE // _CHUNK


def _sc_body(gt_hbm, est_hbm, ig0_hbm, ig1_hbm, ie0_hbm, ie1_hbm, out_hbm,
             idx0, idx1, idx2, idx3, ra, rb, rc, rd, accv, sem):
    wid = lax.axis_index("s") * _NC + lax.axis_index("c")
    base0 = wid * _PER_TILE

    def chunk_body(it, acc):
        base = base0 + it * _CHUNK
        pltpu.sync_copy(ig0_hbm.at[pl.ds(base, _CHUNK)], idx0)
        pltpu.sync_copy(ig1_hbm.at[pl.ds(base, _CHUNK)], idx1)
        pltpu.sync_copy(ie0_hbm.at[pl.ds(base, _CHUNK)], idx2)
        pltpu.sync_copy(ie1_hbm.at[pl.ds(base, _CHUNK)], idx3)
        cps = [
            pltpu.async_copy(gt_hbm.at[idx0], ra, sem),
            pltpu.async_copy(gt_hbm.at[idx1], rb, sem),
            pltpu.async_copy(est_hbm.at[idx2], rc, sem),
            pltpu.async_copy(est_hbm.at[idx3], rd, sem),
        ]
        for cp in cps:
            cp.wait()

        def row_body(r, acc_in):
            for k in range(3):
                sl = pl.ds(k * _LANES, _LANES)
                d = (ra[r, sl] - rb[r, sl]) - (rc[r, sl] - rd[r, sl])
                acc_in = acc_in + d * d
            return acc_in

        return lax.fori_loop(0, _CHUNK, row_body, acc)

    acc = lax.fori_loop(0, _ITERS, chunk_body,
                        jnp.zeros((_LANES,), jnp.float32))
    accv[...] = acc
    pltpu.sync_copy(accv, out_hbm.at[wid])


def kernel(gt_vertices, est_vertices, gt_connections, est_connections):
    B, N, C3 = gt_vertices.shape
    E = gt_connections.shape[0]
    row = C3 * B

    gtT = jnp.transpose(gt_vertices, (1, 2, 0)).reshape(N, row)
    estT = jnp.transpose(est_vertices, (1, 2, 0)).reshape(N, row)
    conn_g = gt_connections.astype(jnp.int32)
    conn_e = est_connections.astype(jnp.int32)
    z = jnp.zeros((_E_PAD - E,), jnp.int32)
    ig0 = jnp.concatenate([conn_g[:, 0], z])
    ig1 = jnp.concatenate([conn_g[:, 1], z])
    ie0 = jnp.concatenate([conn_e[:, 0], z])
    ie1 = jnp.concatenate([conn_e[:, 1], z])

    run = pl.kernel(
        _sc_body,
        mesh=plsc.VectorSubcoreMesh(core_axis_name="c", subcore_axis_name="s"),
        compiler_params=pltpu.CompilerParams(use_tc_tiling_on_sc=False),
        out_type=jax.ShapeDtypeStruct((_NW, _LANES), jnp.float32),
        scratch_types=[
            pltpu.VMEM((_CHUNK,), jnp.int32),
            pltpu.VMEM((_CHUNK,), jnp.int32),
            pltpu.VMEM((_CHUNK,), jnp.int32),
            pltpu.VMEM((_CHUNK,), jnp.int32),
            pltpu.VMEM((_CHUNK, row), jnp.float32),
            pltpu.VMEM((_CHUNK, row), jnp.float32),
            pltpu.VMEM((_CHUNK, row), jnp.float32),
            pltpu.VMEM((_CHUNK, row), jnp.float32),
            pltpu.VMEM((_LANES,), jnp.float32),
            pltpu.SemaphoreType.DMA,
        ],
    )
    partials = run(gtT, estT, ig0, ig1, ie0, ie1)
    return jnp.sum(partials) / (B + 1e-08)


# speedup vs baseline: 14.8401x; 1.6248x over previous
import jax
import jax.numpy as jnp
from jax import lax
from jax.experimental import pallas as pl
from jax.experimental.pallas import tpu as pltpu
from jax.experimental.pallas import tpu_sc as plsc

_LANES = 16
_NC = 2
_NS = 16
_NW = _NC * _NS
_CHUNK = 128
_E_PAD = 819200
_PER_TILE = _E_PAD // _NW
_CHUNKS_PER_TILE = _PER_TILE // _CHUNK
_NCHUNKS = _E_PAD // _CHUNK
_SCC = 20
_NSC = _CHUNKS_PER_TILE // _SCC


def _sc_body(gt_hbm, est_hbm, idx_hbm, out_hbm,
             idxv, ra0, rb0, rc0, rd0, ra1, rb1, rc1, rd1, accv,
             semA, semB):
    wid = lax.axis_index("s") * _NC + lax.axis_index("c")
    chunk0 = wid * _CHUNKS_PER_TILE

    bufs = ((ra0, rb0, rc0, rd0, semA), (ra1, rb1, rc1, rd1, semB))

    def fire(c, b):
        ra_, rb_, rc_, rd_, sem = bufs[b]
        pltpu.async_copy(gt_hbm.at[idxv.at[c, 0]], ra_, sem)
        pltpu.async_copy(gt_hbm.at[idxv.at[c, 1]], rb_, sem)
        pltpu.async_copy(est_hbm.at[idxv.at[c, 2]], rc_, sem)
        pltpu.async_copy(est_hbm.at[idxv.at[c, 3]], rd_, sem)

    def wait(b):
        ra_, rb_, rc_, rd_, sem = bufs[b]
        pltpu.make_async_copy(gt_hbm.at[idxv.at[0, 0]], ra_, sem).wait()
        pltpu.make_async_copy(gt_hbm.at[idxv.at[0, 1]], rb_, sem).wait()
        pltpu.make_async_copy(est_hbm.at[idxv.at[0, 2]], rc_, sem).wait()
        pltpu.make_async_copy(est_hbm.at[idxv.at[0, 3]], rd_, sem).wait()

    def compute(b, accs):
        ra_, rb_, rc_, rd_, _ = bufs[b]

        def row4(rr, accs_in):
            outs = list(accs_in)
            for u in range(4):
                r = rr * 4 + u
                for k in range(3):
                    sl = pl.ds(k * _LANES, _LANES)
                    d = (ra_[r, sl] - rb_[r, sl]) - (rc_[r, sl] - rd_[r, sl])
                    outs[k] = outs[k] + d * d
            return tuple(outs)

        return lax.fori_loop(0, _CHUNK // 4, row4, accs)

    def superchunk(s, accs):
        pltpu.sync_copy(idx_hbm.at[pl.ds(chunk0 + s * _SCC, _SCC)], idxv)
        fire(0, 0)
        fire(1, 1)

        def pair(i2, accs_in):
            wait(0)
            accs_in = compute(0, accs_in)
            fire(2 * i2 + 2, 0)
            wait(1)
            accs_in = compute(1, accs_in)
            fire(2 * i2 + 3, 1)
            return accs_in

        accs = lax.fori_loop(0, _SCC // 2 - 1, pair, accs)
        wait(0)
        accs = compute(0, accs)
        wait(1)
        accs = compute(1, accs)
        return accs

    zero = jnp.zeros((_LANES,), jnp.float32)
    accs = lax.fori_loop(0, _NSC, superchunk, (zero, zero, zero))
    accv[...] = accs[0] + accs[1] + accs[2]
    pltpu.sync_copy(accv, out_hbm.at[wid])


def kernel(gt_vertices, est_vertices, gt_connections, est_connections):
    B, N, C3 = gt_vertices.shape
    E = gt_connections.shape[0]
    row = C3 * B

    gtT = jnp.transpose(gt_vertices, (1, 2, 0)).reshape(N, row)
    estT = jnp.transpose(est_vertices, (1, 2, 0)).reshape(N, row)
    conn_g = gt_connections.astype(jnp.int32)
    conn_e = est_connections.astype(jnp.int32)
    z = jnp.zeros((_E_PAD - E,), jnp.int32)
    ig0 = jnp.concatenate([conn_g[:, 0], z])
    ig1 = jnp.concatenate([conn_g[:, 1], z])
    ie0 = jnp.concatenate([conn_e[:, 0], z])
    ie1 = jnp.concatenate([conn_e[:, 1], z])
    idx_packed = (jnp.stack([ig0, ig1, ie0, ie1])
                  .reshape(4, _NCHUNKS, _CHUNK)
                  .transpose(1, 0, 2))

    run = pl.kernel(
        _sc_body,
        mesh=plsc.VectorSubcoreMesh(core_axis_name="c", subcore_axis_name="s"),
        compiler_params=pltpu.CompilerParams(use_tc_tiling_on_sc=False),
        out_type=jax.ShapeDtypeStruct((_NW, _LANES), jnp.float32),
        scratch_types=[
            pltpu.VMEM((_SCC, 4, _CHUNK), jnp.int32),
            pltpu.VMEM((_CHUNK, row), jnp.float32),
            pltpu.VMEM((_CHUNK, row), jnp.float32),
            pltpu.VMEM((_CHUNK, row), jnp.float32),
            pltpu.VMEM((_CHUNK, row), jnp.float32),
            pltpu.VMEM((_CHUNK, row), jnp.float32),
            pltpu.VMEM((_CHUNK, row), jnp.float32),
            pltpu.VMEM((_CHUNK, row), jnp.float32),
            pltpu.VMEM((_CHUNK, row), jnp.float32),
            pltpu.VMEM((_LANES,), jnp.float32),
            pltpu.SemaphoreType.DMA,
            pltpu.SemaphoreType.DMA,
        ],
    )
    partials = run(gtT, estT, idx_packed)
    return jnp.sum(partials) / (B + 1e-08)


# speedup vs baseline: 15.9316x; 1.0735x over previous
import jax
import jax.numpy as jnp
from jax import lax
from jax.experimental import pallas as pl
from jax.experimental.pallas import tpu as pltpu
from jax.experimental.pallas import tpu_sc as plsc

_LANES = 16
_NC = 2
_NS = 16
_NW = _NC * _NS
_CHUNK = 128
_E_PAD = 819200
_PER_TILE = _E_PAD // _NW
_CHUNKS_PER_TILE = _PER_TILE // _CHUNK
_NCHUNKS = _E_PAD // _CHUNK
_SCC = 20
_CHUNKS_FAST = 300
_CHUNKS_SLOW = 100


def _sc_body(gt_hbm, est_hbm, idx_hbm, out_hbm,
             idxv, ra0, rb0, rc0, rd0, ra1, rb1, rc1, rd1, accv,
             semA, semB):
    cid = lax.axis_index("c")
    sid = lax.axis_index("s")
    wid = sid * _NC + cid
    chunk0 = jnp.where(cid == 0, sid * _CHUNKS_FAST,
                       _NS * _CHUNKS_FAST + sid * _CHUNKS_SLOW)
    nsc = jnp.where(cid == 0, _CHUNKS_FAST // _SCC, _CHUNKS_SLOW // _SCC)

    bufs = ((ra0, rb0, rc0, rd0, semA), (ra1, rb1, rc1, rd1, semB))

    def fire(c, b):
        ra_, rb_, rc_, rd_, sem = bufs[b]
        pltpu.async_copy(gt_hbm.at[idxv.at[c, 0]], ra_, sem)
        pltpu.async_copy(gt_hbm.at[idxv.at[c, 1]], rb_, sem)
        pltpu.async_copy(est_hbm.at[idxv.at[c, 2]], rc_, sem)
        pltpu.async_copy(est_hbm.at[idxv.at[c, 3]], rd_, sem)

    def wait(b):
        ra_, rb_, rc_, rd_, sem = bufs[b]
        pltpu.make_async_copy(gt_hbm.at[idxv.at[0, 0]], ra_, sem).wait()
        pltpu.make_async_copy(gt_hbm.at[idxv.at[0, 1]], rb_, sem).wait()
        pltpu.make_async_copy(est_hbm.at[idxv.at[0, 2]], rc_, sem).wait()
        pltpu.make_async_copy(est_hbm.at[idxv.at[0, 3]], rd_, sem).wait()

    def compute(b, accs):
        ra_, rb_, rc_, rd_, _ = bufs[b]

        def row4(rr, accs_in):
            outs = list(accs_in)
            for u in range(4):
                r = rr * 4 + u
                for k in range(3):
                    sl = pl.ds(k * _LANES, _LANES)
                    d = (ra_[r, sl] - rb_[r, sl]) - (rc_[r, sl] - rd_[r, sl])
                    outs[k] = outs[k] + d * d
            return tuple(outs)

        return lax.fori_loop(0, _CHUNK // 4, row4, accs)

    def superchunk(s, accs):
        pltpu.sync_copy(idx_hbm.at[pl.ds(chunk0 + s * _SCC, _SCC)], idxv)
        fire(0, 0)
        fire(1, 1)

        def pair(i2, accs_in):
            wait(0)
            accs_in = compute(0, accs_in)
            fire(2 * i2 + 2, 0)
            wait(1)
            accs_in = compute(1, accs_in)
            fire(2 * i2 + 3, 1)
            return accs_in

        accs = lax.fori_loop(0, _SCC // 2 - 1, pair, accs)
        wait(0)
        accs = compute(0, accs)
        wait(1)
        accs = compute(1, accs)
        return accs

    zero = jnp.zeros((_LANES,), jnp.float32)
    accs = lax.fori_loop(0, nsc, superchunk, (zero, zero, zero))
    accv[...] = accs[0] + accs[1] + accs[2]
    pltpu.sync_copy(accv, out_hbm.at[wid])


def kernel(gt_vertices, est_vertices, gt_connections, est_connections):
    B, N, C3 = gt_vertices.shape
    E = gt_connections.shape[0]
    row = C3 * B

    gtT = jnp.transpose(gt_vertices, (1, 2, 0)).reshape(N, row)
    estT = jnp.transpose(est_vertices, (1, 2, 0)).reshape(N, row)
    conn_g = gt_connections.astype(jnp.int32)
    conn_e = est_connections.astype(jnp.int32)
    z = jnp.zeros((_E_PAD - E,), jnp.int32)
    ig0 = jnp.concatenate([conn_g[:, 0], z])
    ig1 = jnp.concatenate([conn_g[:, 1], z])
    ie0 = jnp.concatenate([conn_e[:, 0], z])
    ie1 = jnp.concatenate([conn_e[:, 1], z])
    idx_packed = (jnp.stack([ig0, ig1, ie0, ie1])
                  .reshape(4, _NCHUNKS, _CHUNK)
                  .transpose(1, 0, 2))

    run = pl.kernel(
        _sc_body,
        mesh=plsc.VectorSubcoreMesh(core_axis_name="c", subcore_axis_name="s"),
        compiler_params=pltpu.CompilerParams(use_tc_tiling_on_sc=False),
        out_type=jax.ShapeDtypeStruct((_NW, _LANES), jnp.float32),
        scratch_types=[
            pltpu.VMEM((_SCC, 4, _CHUNK), jnp.int32),
            pltpu.VMEM((_CHUNK, row), jnp.float32),
            pltpu.VMEM((_CHUNK, row), jnp.float32),
            pltpu.VMEM((_CHUNK, row), jnp.float32),
            pltpu.VMEM((_CHUNK, row), jnp.float32),
            pltpu.VMEM((_CHUNK, row), jnp.float32),
            pltpu.VMEM((_CHUNK, row), jnp.float32),
            pltpu.VMEM((_CHUNK, row), jnp.float32),
            pltpu.VMEM((_CHUNK, row), jnp.float32),
            pltpu.VMEM((_LANES,), jnp.float32),
            pltpu.SemaphoreType.DMA,
            pltpu.SemaphoreType.DMA,
        ],
    )
    partials = run(gtT, estT, idx_packed)
    return jnp.sum(partials) / (B + 1e-08)
